# Initial kernel scaffold; baseline (speedup 1.0000x reference)
import jax
import jax.numpy as jnp
from jax.experimental import pallas as pl


def kernel(x, edge_index, Wq1, bq1, Wk1, bk1, Wv1, bv1, Ws1, bs1, Wq2, bq2, Wk2, bk2, Wv2, bv2, Ws2, bs2):
    raise NotImplementedError("write your pallas kernel here")


# speedup vs baseline: 25.6070x; 25.6070x over previous
import functools

import jax
import jax.numpy as jnp
from jax import lax
from jax.experimental import pallas as pl
from jax.experimental.pallas import tpu as pltpu
from jax.experimental.pallas import tpu_sc as plsc

N = 50000
E = 800000
NP = 50176
NTILES = 32
C = 128
EPT = 25088
NCHUNK = EPT // C
EP = EPT * NTILES
ROWS_PER_TILE = NP // 16
BK2 = 512

_f32 = jnp.float32
_i32 = jnp.int32


BK1 = 1024
NB1 = NP // BK1


def _k1a_body(x_ref, xm2_ref, mx_ref):
    i = pl.program_id(0)
    x0 = x_ref[:, 0:1]
    x1 = x_ref[:, 1:2]
    bm = jnp.max(x0 * x0 + x1 * x1)

    @pl.when(i == 0)
    def _():
        mx_ref[0] = bm

    mx_ref[0] = jnp.maximum(mx_ref[0], bm)

    @pl.when(i == NB1 - 1)
    def _():
        xm2_ref[...] = jnp.full((1, 1), mx_ref[0], _f32)


def _k1a(xp):
    return pl.pallas_call(
        _k1a_body,
        grid=(NB1,),
        in_specs=[pl.BlockSpec((BK1, 2), lambda i: (i, 0))],
        out_specs=pl.BlockSpec((1, 1), lambda i: (0, 0)),
        out_shape=jax.ShapeDtypeStruct((1, 1), _f32),
        scratch_shapes=[pltpu.SMEM((1,), _f32)],
    )(xp)


def _k1b_body(x_ref, wq_ref, wk_ref, bq_ref, xm2_ref, tab_ref):
    inv_d = 1.0 / 16.0
    A = lax.dot_general(wq_ref[...], wk_ref[...], (((1,), (1,)), ((), ())),
                        preferred_element_type=_f32) * inv_d
    w = lax.dot_general(wk_ref[...], bq_ref[...], (((1,), (1,)), ((), ())),
                        preferred_element_type=_f32) * inv_d
    x0 = x_ref[:, 0:1]
    x1 = x_ref[:, 1:2]
    xmax = jnp.sqrt(xm2_ref[0, 0])
    g0 = x0 * A[0, 0] + x1 * A[1, 0] + w[0, 0]
    g1 = x0 * A[0, 1] + x1 * A[1, 1] + w[1, 0]
    b = jnp.sqrt(g0 * g0 + g1 * g1) * xmax
    tab_ref[...] = jnp.concatenate(
        [x0, x1, x0, x1, g0, g1, b, jnp.zeros_like(b)], axis=1)


def _k1(xp, wq1, wk1, bq1):
    xm2 = _k1a(xp)
    return pl.pallas_call(
        _k1b_body,
        grid=(NB1,),
        in_specs=[
            pl.BlockSpec((BK1, 2), lambda i: (i, 0)),
            pl.BlockSpec((2, 256), lambda i: (0, 0)),
            pl.BlockSpec((2, 256), lambda i: (0, 0)),
            pl.BlockSpec((1, 256), lambda i: (0, 0)),
            pl.BlockSpec((1, 1), lambda i: (0, 0)),
        ],
        out_specs=pl.BlockSpec((BK1, 8), lambda i: (i, 0)),
        out_shape=jax.ShapeDtypeStruct((NP, 8), _f32),
    )(xp, wq1, wk1, bq1, xm2)


def _sc_edge_body(tab, src_idx, dst_idx, zrows, out_parts,
                  idx_s, idx_d, rows_s, rows_d, payload, accum, sem):
    cid = lax.axis_index("c")
    sid = lax.axis_index("s")
    tid = cid * jnp.int32(16) + sid

    row0 = sid * jnp.int32(ROWS_PER_TILE)
    pltpu.sync_copy(zrows, accum.at[pl.ds(row0, ROWS_PER_TILE)])

    lane = jnp.arange(16, dtype=_i32)
    zero16 = jnp.zeros((16,), _f32)
    cols = [jnp.full((16,), c, _i32) for c in range(8)]

    for j in range(C // 16):
        r16 = lane + (j * 16)
        for c in range(3, 8):
            plsc.store_scatter(payload, [r16, cols[c]], zero16)

    plsc.subcore_barrier()

    base = tid * jnp.int32(EPT)

    def chunk(gi, carry):
        off = base + gi * jnp.int32(C)
        pltpu.sync_copy(src_idx.at[pl.ds(off, C)], idx_s)
        pltpu.sync_copy(dst_idx.at[pl.ds(off, C)], idx_d)
        pltpu.async_copy(tab.at[idx_s], rows_s, sem).wait()
        pltpu.async_copy(tab.at[idx_d], rows_d, sem).wait()
        for j in range(C // 16):
            r16 = lane + (j * 16)
            c0 = plsc.load_gather(rows_s, [r16, cols[0]])
            c1 = plsc.load_gather(rows_s, [r16, cols[1]])
            c2 = plsc.load_gather(rows_s, [r16, cols[2]])
            c3 = plsc.load_gather(rows_s, [r16, cols[3]])
            g0 = plsc.load_gather(rows_d, [r16, cols[4]])
            g1 = plsc.load_gather(rows_d, [r16, cols[5]])
            b = plsc.load_gather(rows_d, [r16, cols[6]])
            e = jnp.exp(g0 * c0 + g1 * c1 - b)
            plsc.store_scatter(payload, [r16, cols[0]], e)
            plsc.store_scatter(payload, [r16, cols[1]], e * c2)
            plsc.store_scatter(payload, [r16, cols[2]], e * c3)
        pltpu.sync_copy(payload, accum.at[idx_d], add=True)
        return carry

    lax.fori_loop(jnp.int32(0), jnp.int32(NCHUNK), chunk, jnp.int32(0))
    plsc.subcore_barrier()
    pltpu.sync_copy(accum.at[pl.ds(row0, ROWS_PER_TILE)],
                    out_parts.at[cid, pl.ds(row0, ROWS_PER_TILE)])


@functools.cache
def _sc_edge_kernel():
    return pl.kernel(
        _sc_edge_body,
        out_type=jax.ShapeDtypeStruct((2, NP, 8), _f32),
        mesh=plsc.VectorSubcoreMesh(core_axis_name="c", subcore_axis_name="s"),
        scratch_types=[
            pltpu.VMEM((C,), _i32),
            pltpu.VMEM((C,), _i32),
            pltpu.VMEM((C, 8), _f32),
            pltpu.VMEM((C, 8), _f32),
            pltpu.VMEM((C, 8), _f32),
            pltpu.VMEM_SHARED((NP, 8), _f32),
            pltpu.SemaphoreType.DMA,
        ],
        compiler_params=pltpu.CompilerParams(needs_layout_passes=False,
                                             use_tc_tiling_on_sc=False),
    )


def _sc_edge(*args):
    return _sc_edge_kernel()(*args)


def _k2_body(x_ref, parts_ref, w1_ref, bs1_ref, w2_ref, b2_ref,
             p8_ref, k2m2_ref, mx_ref):
    i = pl.program_id(0)

    a = parts_ref[0] + parts_ref[1]
    z = a[:, 0:1]
    has = z > 0.0
    zs = jnp.where(has, z, 1.0)
    aggx = jnp.where(has, a[:, 1:2] / zs, 0.0)
    aggy = jnp.where(has, a[:, 2:3] / zs, 0.0)
    s = jnp.where(has, 1.0, 0.0).astype(_f32)
    feat = jnp.concatenate(
        [x_ref[...], aggx, aggy, s, jnp.zeros((BK2, 3), _f32)], axis=1)
    h = jnp.maximum(
        jnp.dot(feat, w1_ref[...], preferred_element_type=_f32)
        + bs1_ref[...], 0.0)
    p = jnp.dot(h, w2_ref[...], preferred_element_type=_f32) + b2_ref[...]
    p8_ref[...] = p

    rows = i * BK2 + lax.broadcasted_iota(_i32, (BK2, 1), 0)
    k2n2 = p[:, 2:3] ** 2 + p[:, 3:4] ** 2
    bm = jnp.max(jnp.where(rows < N, k2n2, 0.0))

    @pl.when(i == 0)
    def _():
        mx_ref[0] = bm

    mx_ref[0] = jnp.maximum(mx_ref[0], bm)

    @pl.when(i == (NP // BK2) - 1)
    def _():
        k2m2_ref[...] = jnp.full((1, 1), mx_ref[0], _f32)


def _k2(xp, parts1, w1cat, bs1, w2cat, b2cat):
    nsteps = NP // BK2
    return pl.pallas_call(
        _k2_body,
        grid=(nsteps,),
        in_specs=[
            pl.BlockSpec((BK2, 2), lambda i: (i, 0)),
            pl.BlockSpec((2, BK2, 8), lambda i: (0, i, 0)),
            pl.BlockSpec((8, 256), lambda i: (0, 0)),
            pl.BlockSpec((1, 256), lambda i: (0, 0)),
            pl.BlockSpec((256, 8), lambda i: (0, 0)),
            pl.BlockSpec((1, 8), lambda i: (0, 0)),
        ],
        out_specs=[
            pl.BlockSpec((BK2, 8), lambda i: (i, 0)),
            pl.BlockSpec((1, 1), lambda i: (0, 0)),
        ],
        out_shape=[
            jax.ShapeDtypeStruct((NP, 8), _f32),
            jax.ShapeDtypeStruct((1, 1), _f32),
        ],
        scratch_shapes=[pltpu.SMEM((1,), _f32)],
    )(xp, parts1, w1cat, bs1, w2cat, b2cat)


def _k2b_body(p8_ref, k2m2_ref, tab_ref):
    isr2 = 0.70710678118654752440
    q0 = p8_ref[:, 0:1]
    q1 = p8_ref[:, 1:2]
    k2max = jnp.sqrt(k2m2_ref[0, 0])
    g0 = q0 * isr2
    g1 = q1 * isr2
    b = jnp.sqrt(q0 * q0 + q1 * q1) * (k2max * isr2)
    tab_ref[...] = jnp.concatenate(
        [p8_ref[:, 2:6], g0, g1, b, jnp.zeros_like(b)], axis=1)


def _k2b(p8, k2m2):
    return pl.pallas_call(
        _k2b_body,
        grid=(NB1,),
        in_specs=[pl.BlockSpec((BK1, 8), lambda i: (i, 0)),
                  pl.BlockSpec((1, 1), lambda i: (0, 0))],
        out_specs=pl.BlockSpec((BK1, 8), lambda i: (i, 0)),
        out_shape=jax.ShapeDtypeStruct((NP, 8), _f32),
    )(p8, k2m2)


def _k3_body(parts_ref, p8_ref, out_ref):
    a = parts_ref[0] + parts_ref[1]
    z = a[:, 0:1]
    has = z > 0.0
    zs = jnp.where(has, z, 1.0)
    ox = jnp.where(has, a[:, 1:2] / zs, 0.0) + p8_ref[:, 6:7]
    oy = jnp.where(has, a[:, 2:3] / zs, 0.0) + p8_ref[:, 7:8]
    m = jnp.maximum(ox, oy)
    l = m + jnp.log(jnp.exp(ox - m) + jnp.exp(oy - m))
    out_ref[...] = jnp.concatenate([ox - l, oy - l], axis=1)


def _k3(parts2, p8):
    return pl.pallas_call(
        _k3_body,
        grid=(NB1,),
        in_specs=[pl.BlockSpec((2, BK1, 8), lambda i: (0, i, 0)),
                  pl.BlockSpec((BK1, 8), lambda i: (i, 0))],
        out_specs=pl.BlockSpec((BK1, 2), lambda i: (i, 0)),
        out_shape=jax.ShapeDtypeStruct((N, 2), _f32),
    )(parts2, p8)


def kernel(x, edge_index, Wq1, bq1, Wk1, bk1, Wv1, bv1, Ws1, bs1,
           Wq2, bq2, Wk2, bk2, Wv2, bv2, Ws2, bs2):
    src64 = edge_index[0]
    dst64 = edge_index[1]
    with jax.enable_x64(False):
        return _kernel32(x, src64.astype(_i32), dst64.astype(_i32),
                         Wq1, bq1, Wk1, bk1, Wv1, bv1, Ws1, bs1,
                         Wq2, bq2, Wk2, bk2, Wv2, bv2, Ws2, bs2)


def _kernel32(x, src, dst, Wq1, bq1, Wk1, bk1, Wv1, bv1, Ws1, bs1,
              Wq2, bq2, Wk2, bk2, Wv2, bv2, Ws2, bs2):
    x = x.astype(_f32)
    xp = jnp.pad(x, ((0, NP - N), (0, 0)))

    padi = jnp.full((EP - E,), N, _i32)
    srcp = jnp.concatenate([src, padi])
    dstp = jnp.concatenate([dst, padi])

    zrows = jnp.zeros((ROWS_PER_TILE, 8), _f32)

    tab1 = _k1(xp, Wq1.astype(_f32), Wk1.astype(_f32),
               bq1.astype(_f32)[None, :])
    parts1 = _sc_edge(tab1, srcp, dstp, zrows)

    w1cat = jnp.concatenate(
        [Ws1, Wv1, bv1[None, :], jnp.zeros((3, 256), _f32)], axis=0)
    w2cat = jnp.concatenate([Wq2, Wk2, Wv2, Ws2], axis=1)
    b2cat = jnp.concatenate([bq2, bk2, bv2, bs2])[None, :]

    p8, k2m2 = _k2(xp, parts1, w1cat.astype(_f32),
                   bs1.astype(_f32)[None, :], w2cat.astype(_f32),
                   b2cat.astype(_f32))

    tab2 = _k2b(p8, k2m2)

    parts2 = _sc_edge(tab2, srcp, dstp, zrows)

    return _k3(parts2, p8)


# speedup vs baseline: 53.6013x; 2.0932x over previous
import functools

import jax
import jax.numpy as jnp
from jax import lax
from jax.experimental import pallas as pl
from jax.experimental.pallas import tpu as pltpu
from jax.experimental.pallas import tpu_sc as plsc

N = 50000
E = 800000
NP = 50176
NTILES = 32
C = 128
EPT = 25088
NCHUNK = EPT // C
EP = EPT * NTILES
ROWS_PER_TILE = NP // 16
BK2 = 512

_f32 = jnp.float32
_i32 = jnp.int32


BK1 = 1024
NB1 = NP // BK1


def _k1a_body(x_ref, xm2_ref, mx_ref):
    i = pl.program_id(0)
    x0 = x_ref[:, 0:1]
    x1 = x_ref[:, 1:2]
    bm = jnp.max(x0 * x0 + x1 * x1)

    @pl.when(i == 0)
    def _():
        mx_ref[0] = bm

    mx_ref[0] = jnp.maximum(mx_ref[0], bm)

    @pl.when(i == NB1 - 1)
    def _():
        xm2_ref[...] = jnp.full((1, 1), mx_ref[0], _f32)


def _k1a(xp):
    return pl.pallas_call(
        _k1a_body,
        grid=(NB1,),
        in_specs=[pl.BlockSpec((BK1, 2), lambda i: (i, 0))],
        out_specs=pl.BlockSpec((1, 1), lambda i: (0, 0)),
        out_shape=jax.ShapeDtypeStruct((1, 1), _f32),
        scratch_shapes=[pltpu.SMEM((1,), _f32)],
    )(xp)


def _k1b_body(x_ref, wq_ref, wk_ref, bq_ref, xm2_ref, tab_ref):
    inv_d = 1.0 / 16.0
    A = lax.dot_general(wq_ref[...], wk_ref[...], (((1,), (1,)), ((), ())),
                        preferred_element_type=_f32) * inv_d
    w = lax.dot_general(wk_ref[...], bq_ref[...], (((1,), (1,)), ((), ())),
                        preferred_element_type=_f32) * inv_d
    x0 = x_ref[:, 0:1]
    x1 = x_ref[:, 1:2]
    xmax = jnp.sqrt(xm2_ref[0, 0])
    g0 = x0 * A[0, 0] + x1 * A[1, 0] + w[0, 0]
    g1 = x0 * A[0, 1] + x1 * A[1, 1] + w[1, 0]
    b = jnp.sqrt(g0 * g0 + g1 * g1) * xmax
    tab_ref[...] = jnp.concatenate(
        [x0, x1, x0, x1, g0, g1, b, jnp.zeros_like(b)], axis=1)


def _k1(xp, wq1, wk1, bq1):
    xm2 = _k1a(xp)
    return pl.pallas_call(
        _k1b_body,
        grid=(NB1,),
        in_specs=[
            pl.BlockSpec((BK1, 2), lambda i: (i, 0)),
            pl.BlockSpec((2, 256), lambda i: (0, 0)),
            pl.BlockSpec((2, 256), lambda i: (0, 0)),
            pl.BlockSpec((1, 256), lambda i: (0, 0)),
            pl.BlockSpec((1, 1), lambda i: (0, 0)),
        ],
        out_specs=pl.BlockSpec((BK1, 8), lambda i: (i, 0)),
        out_shape=jax.ShapeDtypeStruct((NP, 8), _f32),
    )(xp, wq1, wk1, bq1, xm2)


NB = 4


def _sc_edge_body(tab, src_idx, dst_idx, zrows, out_parts,
                  siv, div, rows_s, rows_d, pays, gsem, ssem):
    cid = lax.axis_index("c")
    sid = lax.axis_index("s")
    tid = cid * jnp.int32(16) + sid

    accum = pays[NB]
    pays = pays[:NB]

    row0 = sid * jnp.int32(ROWS_PER_TILE)
    pltpu.sync_copy(zrows, accum.at[pl.ds(row0, ROWS_PER_TILE)])

    pltpu.sync_copy(src_idx.at[tid], siv)
    pltpu.sync_copy(dst_idx.at[tid], div)

    lane = jnp.arange(16, dtype=_i32)
    zero16 = jnp.zeros((16,), _f32)
    cols = [jnp.full((16,), c, _i32) for c in range(8)]

    for b in range(NB):
        for j in range(C // 16):
            r16 = lane + (j * 16)
            for c in range(3, 8):
                plsc.store_scatter(pays[b], [r16, cols[c]], zero16)

    plsc.subcore_barrier()

    def g_start(g, b):
        pltpu.async_copy(tab.at[siv.at[g, 0]], rows_s[b], gsem[b])
        pltpu.async_copy(tab.at[div.at[g, 0]], rows_d[b], gsem[b])

    def g_wait(g, b):
        pltpu.make_async_copy(tab.at[siv.at[g, 0]], rows_s[b], gsem[b]).wait()
        pltpu.make_async_copy(tab.at[div.at[g, 0]], rows_d[b], gsem[b]).wait()

    def s_start(g, b):
        pltpu.async_copy(pays[b], accum.at[div.at[g, 0]], ssem[b], add=True)

    def s_wait(g, b):
        pltpu.make_async_copy(pays[b], accum.at[div.at[g, 0]],
                              ssem[b]).wait()

    for b in range(NB):
        g_start(jnp.int32(b), b)

    def outer(i, carry):
        g0 = i * jnp.int32(NB)
        for b in range(NB):
            g = g0 + jnp.int32(b)
            g_wait(g, b)

            @pl.when(i > 0)
            def _():
                s_wait(g, b)

            for j in range(C // 16):
                r16 = lane + (j * 16)
                c0 = plsc.load_gather(rows_s[b], [r16, cols[0]])
                c1 = plsc.load_gather(rows_s[b], [r16, cols[1]])
                c2 = plsc.load_gather(rows_s[b], [r16, cols[2]])
                c3 = plsc.load_gather(rows_s[b], [r16, cols[3]])
                g0v = plsc.load_gather(rows_d[b], [r16, cols[4]])
                g1v = plsc.load_gather(rows_d[b], [r16, cols[5]])
                bb = plsc.load_gather(rows_d[b], [r16, cols[6]])
                e = jnp.exp(g0v * c0 + g1v * c1 - bb)
                plsc.store_scatter(pays[b], [r16, cols[0]], e)
                plsc.store_scatter(pays[b], [r16, cols[1]], e * c2)
                plsc.store_scatter(pays[b], [r16, cols[2]], e * c3)
            s_start(g, b)
            gnext = jnp.minimum(g + jnp.int32(NB), jnp.int32(NCHUNK - 1))
            g_start(gnext, b)
        return carry

    lax.fori_loop(jnp.int32(0), jnp.int32(NCHUNK // NB), outer, jnp.int32(0))

    for b in range(NB):
        g_wait(jnp.int32(NCHUNK - 1), b)
        s_wait(jnp.int32(NCHUNK - NB + b), b)

    plsc.subcore_barrier()
    pltpu.sync_copy(accum.at[pl.ds(row0, ROWS_PER_TILE)],
                    out_parts.at[cid, pl.ds(row0, ROWS_PER_TILE)])


@functools.cache
def _sc_edge_kernel():
    return pl.kernel(
        _sc_edge_body,
        out_type=jax.ShapeDtypeStruct((2, NP, 8), _f32),
        mesh=plsc.VectorSubcoreMesh(core_axis_name="c", subcore_axis_name="s"),
        scratch_types=[
            pltpu.VMEM((NCHUNK, 1, C), _i32),
            pltpu.VMEM((NCHUNK, 1, C), _i32),
            [pltpu.VMEM((C, 8), _f32) for _ in range(NB)],
            [pltpu.VMEM((C, 8), _f32) for _ in range(NB)],
            [pltpu.VMEM((C, 8), _f32) for _ in range(NB)]
            + [pltpu.VMEM_SHARED((NP, 8), _f32)],
            [pltpu.SemaphoreType.DMA for _ in range(NB)],
            [pltpu.SemaphoreType.DMA for _ in range(NB)],
        ],
        compiler_params=pltpu.CompilerParams(needs_layout_passes=False,
                                             use_tc_tiling_on_sc=False),
    )


def _sc_edge(*args):
    return _sc_edge_kernel()(*args)


def _k2_body(x_ref, parts_ref, w1_ref, bs1_ref, w2_ref, b2_ref,
             p8_ref, k2m2_ref, mx_ref):
    i = pl.program_id(0)

    a = parts_ref[0] + parts_ref[1]
    z = a[:, 0:1]
    has = z > 0.0
    zs = jnp.where(has, z, 1.0)
    aggx = jnp.where(has, a[:, 1:2] / zs, 0.0)
    aggy = jnp.where(has, a[:, 2:3] / zs, 0.0)
    s = jnp.where(has, 1.0, 0.0).astype(_f32)
    feat = jnp.concatenate(
        [x_ref[...], aggx, aggy, s, jnp.zeros((BK2, 3), _f32)], axis=1)
    h = jnp.maximum(
        jnp.dot(feat, w1_ref[...], preferred_element_type=_f32)
        + bs1_ref[...], 0.0)
    p = jnp.dot(h, w2_ref[...], preferred_element_type=_f32) + b2_ref[...]
    p8_ref[...] = p

    rows = i * BK2 + lax.broadcasted_iota(_i32, (BK2, 1), 0)
    k2n2 = p[:, 2:3] ** 2 + p[:, 3:4] ** 2
    bm = jnp.max(jnp.where(rows < N, k2n2, 0.0))

    @pl.when(i == 0)
    def _():
        mx_ref[0] = bm

    mx_ref[0] = jnp.maximum(mx_ref[0], bm)

    @pl.when(i == (NP // BK2) - 1)
    def _():
        k2m2_ref[...] = jnp.full((1, 1), mx_ref[0], _f32)


def _k2(xp, parts1, w1cat, bs1, w2cat, b2cat):
    nsteps = NP // BK2
    return pl.pallas_call(
        _k2_body,
        grid=(nsteps,),
        in_specs=[
            pl.BlockSpec((BK2, 2), lambda i: (i, 0)),
            pl.BlockSpec((2, BK2, 8), lambda i: (0, i, 0)),
            pl.BlockSpec((8, 256), lambda i: (0, 0)),
            pl.BlockSpec((1, 256), lambda i: (0, 0)),
            pl.BlockSpec((256, 8), lambda i: (0, 0)),
            pl.BlockSpec((1, 8), lambda i: (0, 0)),
        ],
        out_specs=[
            pl.BlockSpec((BK2, 8), lambda i: (i, 0)),
            pl.BlockSpec((1, 1), lambda i: (0, 0)),
        ],
        out_shape=[
            jax.ShapeDtypeStruct((NP, 8), _f32),
            jax.ShapeDtypeStruct((1, 1), _f32),
        ],
        scratch_shapes=[pltpu.SMEM((1,), _f32)],
    )(xp, parts1, w1cat, bs1, w2cat, b2cat)


def _k2b_body(p8_ref, k2m2_ref, tab_ref):
    isr2 = 0.70710678118654752440
    q0 = p8_ref[:, 0:1]
    q1 = p8_ref[:, 1:2]
    k2max = jnp.sqrt(k2m2_ref[0, 0])
    g0 = q0 * isr2
    g1 = q1 * isr2
    b = jnp.sqrt(q0 * q0 + q1 * q1) * (k2max * isr2)
    tab_ref[...] = jnp.concatenate(
        [p8_ref[:, 2:6], g0, g1, b, jnp.zeros_like(b)], axis=1)


def _k2b(p8, k2m2):
    return pl.pallas_call(
        _k2b_body,
        grid=(NB1,),
        in_specs=[pl.BlockSpec((BK1, 8), lambda i: (i, 0)),
                  pl.BlockSpec((1, 1), lambda i: (0, 0))],
        out_specs=pl.BlockSpec((BK1, 8), lambda i: (i, 0)),
        out_shape=jax.ShapeDtypeStruct((NP, 8), _f32),
    )(p8, k2m2)


def _k3_body(parts_ref, p8_ref, out_ref):
    a = parts_ref[0] + parts_ref[1]
    z = a[:, 0:1]
    has = z > 0.0
    zs = jnp.where(has, z, 1.0)
    ox = jnp.where(has, a[:, 1:2] / zs, 0.0) + p8_ref[:, 6:7]
    oy = jnp.where(has, a[:, 2:3] / zs, 0.0) + p8_ref[:, 7:8]
    m = jnp.maximum(ox, oy)
    l = m + jnp.log(jnp.exp(ox - m) + jnp.exp(oy - m))
    out_ref[...] = jnp.concatenate([ox - l, oy - l], axis=1)


def _k3(parts2, p8):
    return pl.pallas_call(
        _k3_body,
        grid=(NB1,),
        in_specs=[pl.BlockSpec((2, BK1, 8), lambda i: (0, i, 0)),
                  pl.BlockSpec((BK1, 8), lambda i: (i, 0))],
        out_specs=pl.BlockSpec((BK1, 2), lambda i: (i, 0)),
        out_shape=jax.ShapeDtypeStruct((N, 2), _f32),
    )(parts2, p8)


def kernel(x, edge_index, Wq1, bq1, Wk1, bk1, Wv1, bv1, Ws1, bs1,
           Wq2, bq2, Wk2, bk2, Wv2, bv2, Ws2, bs2):
    src64 = edge_index[0]
    dst64 = edge_index[1]
    with jax.enable_x64(False):
        return _kernel32(x, src64.astype(_i32), dst64.astype(_i32),
                         Wq1, bq1, Wk1, bk1, Wv1, bv1, Ws1, bs1,
                         Wq2, bq2, Wk2, bk2, Wv2, bv2, Ws2, bs2)


def _kernel32(x, src, dst, Wq1, bq1, Wk1, bk1, Wv1, bv1, Ws1, bs1,
              Wq2, bq2, Wk2, bk2, Wv2, bv2, Ws2, bs2):
    x = x.astype(_f32)
    xp = jnp.pad(x, ((0, NP - N), (0, 0)))

    padi = jnp.full((EP - E,), N, _i32)
    srcp = jnp.concatenate([src, padi]).reshape(NTILES, NCHUNK, 1, C)
    dstp = jnp.concatenate([dst, padi]).reshape(NTILES, NCHUNK, 1, C)

    zrows = jnp.zeros((ROWS_PER_TILE, 8), _f32)

    tab1 = _k1(xp, Wq1.astype(_f32), Wk1.astype(_f32),
               bq1.astype(_f32)[None, :])
    parts1 = _sc_edge(tab1, srcp, dstp, zrows)

    w1cat = jnp.concatenate(
        [Ws1, Wv1, bv1[None, :], jnp.zeros((3, 256), _f32)], axis=0)
    w2cat = jnp.concatenate([Wq2, Wk2, Wv2, Ws2], axis=1)
    b2cat = jnp.concatenate([bq2, bk2, bv2, bs2])[None, :]

    p8, k2m2 = _k2(xp, parts1, w1cat.astype(_f32),
                   bs1.astype(_f32)[None, :], w2cat.astype(_f32),
                   b2cat.astype(_f32))

    tab2 = _k2b(p8, k2m2)

    parts2 = _sc_edge(tab2, srcp, dstp, zrows)

    return _k3(parts2, p8)
